# baseline (speedup 1.0000x reference)
import functools

import jax
import jax.numpy as jnp
from jax import lax
from jax.experimental import pallas as pl
from jax.experimental.pallas import tpu as pltpu
from jax.experimental.pallas import tpu_sc as plsc

VOCAB = 1000000
EMBED = 64
BATCH = 16384

NUM_CORES = 2
NUM_SUBCORES = 16
NW = NUM_CORES * NUM_SUBCORES
E_PER_W = EMBED // NW

LOG2W = 14
WIN = 1 << LOG2W
NWIN = (VOCAB + WIN - 1) // WIN
LAST_WIN = 512
TAIL = (NWIN - 1) * WIN + LAST_WIN
NTAIL = VOCAB - TAIL
NVEC = BATCH // 16
HALF = BATCH // 2

_DUP_FIRST = 1

_mesh = plsc.VectorSubcoreMesh(core_axis_name="c", subcore_axis_name="s")


@functools.partial(
    pl.kernel,
    mesh=_mesh,
    out_type=jax.ShapeDtypeStruct((EMBED, BATCH), jnp.float32),
    scratch_types=[
        pltpu.VMEM((HALF,), jnp.int32),
        pltpu.VMEM((BATCH + 16,), jnp.int32),
        pltpu.VMEM((64,), jnp.int32),
        pltpu.VMEM((2, E_PER_W, WIN), jnp.float32),
        pltpu.VMEM((E_PER_W, BATCH), jnp.float32),
        pltpu.VMEM((E_PER_W, NTAIL), jnp.float32),
        pltpu.SemaphoreType.DMA,
        pltpu.SemaphoreType.DMA,
    ],
    compiler_params=pltpu.CompilerParams(needs_layout_passes=False),
)
def _gather(idx_hbm, w_hbm, wtail_hbm, out_hbm,
            idx_v, bpk_v, off_v, win_v, out_v, tail_v, sem, wsem):
    wid = lax.axis_index("s") * NUM_CORES + lax.axis_index("c")
    row0 = wid * E_PER_W
    pltpu.sync_copy(wtail_hbm.at[pl.ds(row0, E_PER_W)], tail_v)

    lanes = lax.iota(jnp.int32, 16)
    z16 = jnp.zeros((16,), jnp.int32)
    o16 = jnp.ones((16,), jnp.int32)

    for q in range(4):
        off_v[pl.ds(q * 16, 16)] = z16
    bpk_v[pl.ds(BATCH, 16)] = z16

    def pass_a(j, _):
        ivec = idx_v[pl.ds(j * 16, 16)]
        w = lax.shift_right_logical(ivec, LOG2W)
        dup, last = plsc.scan_count(w)
        plsc.addupdate_scatter(off_v, [w], dup + (1 - _DUP_FIRST), mask=last)
        return ()

    for h in range(2):
        pltpu.sync_copy(idx_hbm.at[pl.ds(h * HALF, HALF)], idx_v)
        lax.fori_loop(0, NVEC // 2, pass_a, ())

    cvecs = [off_v[pl.ds(q * 16, 16)] for q in range(4)]
    bvecs = []
    carry = jnp.int32(0)
    for q in range(4):
        inc = plsc.cumsum(cvecs[q])
        bvecs.append(inc - cvecs[q] + carry)
        carry = carry + inc[15]

    for q in range(4):
        off_v[pl.ds(q * 16, 16)] = bvecs[q]

    for h in range(2):
        pltpu.sync_copy(idx_hbm.at[pl.ds(h * HALF, HALF)], idx_v)

        def pass_b(j, _, h=h):
            ivec = idx_v[pl.ds(j * 16, 16)]
            w = lax.shift_right_logical(ivec, LOG2W)
            cur = plsc.load_gather(off_v, [w])
            dup, last = plsc.scan_count(w)
            pos = cur + (dup - _DUP_FIRST)
            bvec = lanes + (j * 16 + h * HALF)
            pk = jnp.bitwise_or(lax.shift_left(bvec, LOG2W),
                                jnp.bitwise_and(ivec, WIN - 1))
            plsc.store_scatter(bpk_v, [pos], pk)
            plsc.addupdate_scatter(off_v, [w], dup + (1 - _DUP_FIRST), mask=last)
            return ()

        lax.fori_loop(0, NVEC // 2, pass_b, ())

    def start_win(w):
        wsize = WIN if w < NWIN - 1 else LAST_WIN
        return [
            pltpu.async_copy(
                w_hbm.at[pl.ds(row0 + e, 1), pl.ds(w * WIN, wsize)],
                win_v.at[w % 2].at[pl.ds(e, 1), pl.ds(0, wsize)],
                wsem,
            )
            for e in range(E_PER_W)
        ]

    cps = start_win(0)
    for w in range(NWIN):
        for c in cps:
            c.wait()
        if w + 1 < NWIN:
            cps = start_win(w + 1)
        buf = win_v.at[w % 2]
        base = bvecs[w // 16][w % 16]
        end = base + cvecs[w // 16][w % 16]
        jstart = lax.shift_right_logical(base, 4)
        jend = lax.shift_right_logical(end + 15, 4)

        is_last = w == NWIN - 1

        def win_body(j, _, base=base, end=end, is_last=is_last, buf=buf):
            lane_pos = lanes + j * 16
            m = (lane_pos >= base) & (lane_pos < end)
            pk = plsc.load_gather(bpk_v, [lane_pos])
            lofs = jnp.bitwise_and(pk, WIN - 1)
            bp = lax.shift_right_logical(pk, LOG2W)
            if not is_last:
                c0 = plsc.load_gather(buf, [z16, lofs], mask=m)
                c1 = plsc.load_gather(buf, [o16, lofs], mask=m)
            else:
                m_in = m & (lofs < LAST_WIN)
                m_tl = m & (lofs >= LAST_WIN)
                tofs = lofs - LAST_WIN
                c0a = plsc.load_gather(buf, [z16, lofs], mask=m_in)
                c1a = plsc.load_gather(buf, [o16, lofs], mask=m_in)
                c0b = plsc.load_gather(tail_v, [z16, tofs], mask=m_tl)
                c1b = plsc.load_gather(tail_v, [o16, tofs], mask=m_tl)
                c0 = jnp.where(m_in, c0a, c0b)
                c1 = jnp.where(m_in, c1a, c1b)
            plsc.store_scatter(out_v, [z16, bp], c0, mask=m)
            plsc.store_scatter(out_v, [o16, bp], c1, mask=m)
            return ()

        lax.fori_loop(jstart, jend, win_body, ())

    pltpu.sync_copy(out_v, out_hbm.at[pl.ds(row0, E_PER_W)])


def kernel(inputs, v):
    idx = inputs.reshape(BATCH)
    wtail = v[TAIL:, :].T
    out_t = _gather(idx, v.T, wtail)
    return out_t.T


# speedup vs baseline: 1.0195x; 1.0195x over previous
import functools

import jax
import jax.numpy as jnp
from jax import lax
from jax.experimental import pallas as pl
from jax.experimental.pallas import tpu as pltpu
from jax.experimental.pallas import tpu_sc as plsc

VOCAB = 1000000
EMBED = 64
BATCH = 16384

NUM_CORES = 2
NUM_SUBCORES = 16
NW = NUM_CORES * NUM_SUBCORES
E_PER_W = EMBED // NW

LOG2W = 14
WIN = 1 << LOG2W
NWIN = (VOCAB + WIN - 1) // WIN
LAST_WIN = 512
TAIL = (NWIN - 1) * WIN + LAST_WIN
NTAIL = VOCAB - TAIL
NVEC = BATCH // 16
HALF = BATCH // 2

_DUP_FIRST = 1

_mesh = plsc.VectorSubcoreMesh(core_axis_name="c", subcore_axis_name="s")


@functools.partial(
    pl.kernel,
    mesh=_mesh,
    out_type=jax.ShapeDtypeStruct((EMBED, BATCH), jnp.float32),
    scratch_types=[
        pltpu.VMEM((HALF,), jnp.int32),
        pltpu.VMEM((BATCH + 16,), jnp.int32),
        pltpu.VMEM((64,), jnp.int32),
        pltpu.VMEM((2, E_PER_W, WIN), jnp.float32),
        pltpu.VMEM((E_PER_W, BATCH), jnp.float32),
        pltpu.VMEM((E_PER_W, NTAIL), jnp.float32),
        pltpu.SemaphoreType.DMA,
        pltpu.SemaphoreType.DMA,
    ],
    compiler_params=pltpu.CompilerParams(needs_layout_passes=False),
)
def _gather(idx_hbm, w_hbm, wtail_hbm, out_hbm,
            idx_v, bpk_v, off_v, win_v, out_v, tail_v, sem, wsem):
    wid = lax.axis_index("s") * NUM_CORES + lax.axis_index("c")
    row0 = wid * E_PER_W
    pltpu.sync_copy(wtail_hbm.at[pl.ds(row0, E_PER_W)], tail_v)

    lanes = lax.iota(jnp.int32, 16)
    z16 = jnp.zeros((16,), jnp.int32)
    o16 = jnp.ones((16,), jnp.int32)

    for q in range(4):
        off_v[pl.ds(q * 16, 16)] = z16
    bpk_v[pl.ds(BATCH, 16)] = z16

    def pass_a(j, _):
        ivec = idx_v[pl.ds(j * 16, 16)]
        w = lax.shift_right_logical(ivec, LOG2W)
        dup, last = plsc.scan_count(w)
        plsc.addupdate_scatter(off_v, [w], dup + (1 - _DUP_FIRST), mask=last)
        return ()

    for h in range(2):
        pltpu.sync_copy(idx_hbm.at[pl.ds(h * HALF, HALF)], idx_v)
        lax.fori_loop(0, NVEC // 2, pass_a, ())

    cvecs = [off_v[pl.ds(q * 16, 16)] for q in range(4)]
    bvecs = []
    carry = jnp.int32(0)
    for q in range(4):
        inc = plsc.cumsum(cvecs[q])
        bvecs.append(inc - cvecs[q] + carry)
        carry = carry + inc[15]

    for q in range(4):
        off_v[pl.ds(q * 16, 16)] = bvecs[q]

    for h in range(2):
        pltpu.sync_copy(idx_hbm.at[pl.ds(h * HALF, HALF)], idx_v)

        def pass_b(j, _, h=h):
            ivec = idx_v[pl.ds(j * 16, 16)]
            w = lax.shift_right_logical(ivec, LOG2W)
            cur = plsc.load_gather(off_v, [w])
            dup, last = plsc.scan_count(w)
            pos = cur + (dup - _DUP_FIRST)
            bvec = lanes + (j * 16 + h * HALF)
            pk = jnp.bitwise_or(lax.shift_left(bvec, LOG2W),
                                jnp.bitwise_and(ivec, WIN - 1))
            plsc.store_scatter(bpk_v, [pos], pk)
            plsc.addupdate_scatter(off_v, [w], dup + (1 - _DUP_FIRST), mask=last)
            return ()

        lax.fori_loop(0, NVEC // 2, pass_b, ())

    def start_win(w):
        wsize = WIN if w < NWIN - 1 else LAST_WIN
        return pltpu.async_copy(
            w_hbm.at[pl.ds(row0, E_PER_W), pl.ds(w * WIN, wsize)],
            win_v.at[w % 2].at[pl.ds(0, E_PER_W), pl.ds(0, wsize)],
            wsem,
        )

    cp = start_win(0)
    for w in range(NWIN):
        cp.wait()
        if w + 1 < NWIN:
            cp = start_win(w + 1)
        buf = win_v.at[w % 2]
        base = bvecs[w // 16][w % 16]
        end = base + cvecs[w // 16][w % 16]
        jstart = lax.shift_right_logical(base, 4)
        jend = lax.shift_right_logical(end + 15, 4)

        is_last = w == NWIN - 1

        @plsc.parallel_loop(jstart, jend, unroll=2)
        def win_body(j, base=base, end=end, is_last=is_last, buf=buf):
            lane_pos = lanes + j * 16
            m = (lane_pos >= base) & (lane_pos < end)
            pk = plsc.load_gather(bpk_v, [lane_pos])
            lofs = jnp.bitwise_and(pk, WIN - 1)
            bp = lax.shift_right_logical(pk, LOG2W)
            if not is_last:
                c0 = plsc.load_gather(buf, [z16, lofs], mask=m)
                c1 = plsc.load_gather(buf, [o16, lofs], mask=m)
            else:
                m_in = m & (lofs < LAST_WIN)
                m_tl = m & (lofs >= LAST_WIN)
                tofs = lofs - LAST_WIN
                c0a = plsc.load_gather(buf, [z16, lofs], mask=m_in)
                c1a = plsc.load_gather(buf, [o16, lofs], mask=m_in)
                c0b = plsc.load_gather(tail_v, [z16, tofs], mask=m_tl)
                c1b = plsc.load_gather(tail_v, [o16, tofs], mask=m_tl)
                c0 = jnp.where(m_in, c0a, c0b)
                c1 = jnp.where(m_in, c1a, c1b)
            plsc.store_scatter(out_v, [z16, bp], c0, mask=m)
            plsc.store_scatter(out_v, [o16, bp], c1, mask=m)

    pltpu.sync_copy(out_v, out_hbm.at[pl.ds(row0, E_PER_W)])


def kernel(inputs, v):
    idx = inputs.reshape(BATCH)
    wtail = v[TAIL:, :].T
    out_t = _gather(idx, v.T, wtail)
    return out_t.T


# speedup vs baseline: 1.3247x; 1.2995x over previous
import functools

import jax
import jax.numpy as jnp
from jax import lax
from jax.experimental import pallas as pl
from jax.experimental.pallas import tpu as pltpu
from jax.experimental.pallas import tpu_sc as plsc

VOCAB = 1000000
EMBED = 64
BATCH = 16384

NUM_CORES = 2
NUM_SUBCORES = 16
NW = NUM_CORES * NUM_SUBCORES
E_PER_W = EMBED // NW

LOG2W = 14
WIN = 1 << LOG2W
NWIN = (VOCAB + WIN - 1) // WIN
LAST_WIN = 512
TAIL = (NWIN - 1) * WIN + LAST_WIN
NTAIL = VOCAB - TAIL
NVEC = BATCH // 16
HALF = BATCH // 2

_DUP_FIRST = 1

_mesh = plsc.VectorSubcoreMesh(core_axis_name="c", subcore_axis_name="s")


@functools.partial(
    pl.kernel,
    mesh=_mesh,
    out_type=jax.ShapeDtypeStruct((EMBED, BATCH), jnp.float32),
    scratch_types=[
        pltpu.VMEM((HALF,), jnp.int32),
        pltpu.VMEM((BATCH + 16,), jnp.int32),
        pltpu.VMEM((64,), jnp.int32),
        pltpu.VMEM((2, E_PER_W, WIN), jnp.float32),
        pltpu.VMEM((E_PER_W, BATCH), jnp.float32),
        pltpu.VMEM((E_PER_W, NTAIL), jnp.float32),
        pltpu.SemaphoreType.DMA,
        pltpu.SemaphoreType.DMA,
    ],
    compiler_params=pltpu.CompilerParams(needs_layout_passes=False),
)
def _gather(idx_hbm, w_hbm, wtail_hbm, out_hbm,
            idx_v, bpk_v, off_v, win_v, out_v, tail_v, sem, wsem):
    wid = lax.axis_index("s") * NUM_CORES + lax.axis_index("c")
    row0 = wid * E_PER_W
    pltpu.sync_copy(wtail_hbm.at[pl.ds(row0, E_PER_W)], tail_v)

    lanes = lax.iota(jnp.int32, 16)
    z16 = jnp.zeros((16,), jnp.int32)
    o16 = jnp.ones((16,), jnp.int32)

    def start_win(w):
        wsize = WIN if w < NWIN - 1 else LAST_WIN
        return pltpu.async_copy(
            w_hbm.at[pl.ds(row0, E_PER_W), pl.ds(w * WIN, wsize)],
            win_v.at[w % 2].at[pl.ds(0, E_PER_W), pl.ds(0, wsize)],
            wsem,
        )

    cp0 = start_win(0)
    cp1 = start_win(1)

    for q in range(4):
        off_v[pl.ds(q * 16, 16)] = z16
    bpk_v[pl.ds(BATCH, 16)] = z16

    def pass_a(j, _):
        ivec = idx_v[pl.ds(j * 16, 16)]
        w = lax.shift_right_logical(ivec, LOG2W)
        dup, last = plsc.scan_count(w)
        plsc.addupdate_scatter(off_v, [w], dup + (1 - _DUP_FIRST), mask=last)
        return ()

    for h in range(2):
        pltpu.sync_copy(idx_hbm.at[pl.ds(h * HALF, HALF)], idx_v)
        lax.fori_loop(0, NVEC // 2, pass_a, ())

    cvecs = [off_v[pl.ds(q * 16, 16)] for q in range(4)]
    bvecs = []
    carry = jnp.int32(0)
    for q in range(4):
        inc = plsc.cumsum(cvecs[q])
        bvecs.append(inc - cvecs[q] + carry)
        carry = carry + inc[15]

    for q in range(4):
        off_v[pl.ds(q * 16, 16)] = bvecs[q]

    for h in range(2):
        pltpu.sync_copy(idx_hbm.at[pl.ds(h * HALF, HALF)], idx_v)

        def pass_b(j, _, h=h):
            ivec = idx_v[pl.ds(j * 16, 16)]
            w = lax.shift_right_logical(ivec, LOG2W)
            cur = plsc.load_gather(off_v, [w])
            dup, last = plsc.scan_count(w)
            pos = cur + (dup - _DUP_FIRST)
            bvec = lanes + (j * 16 + h * HALF)
            pk = jnp.bitwise_or(lax.shift_left(bvec, LOG2W),
                                jnp.bitwise_and(ivec, WIN - 1))
            plsc.store_scatter(bpk_v, [pos], pk)
            plsc.addupdate_scatter(off_v, [w], dup + (1 - _DUP_FIRST), mask=last)
            return ()

        lax.fori_loop(0, NVEC // 2, pass_b, ())

    cps = [cp0, cp1]
    for w in range(NWIN):
        cps[w].wait()
        buf = win_v.at[w % 2]
        base = bvecs[w // 16][w % 16]
        end = base + cvecs[w // 16][w % 16]
        jstart = lax.shift_right_logical(base, 4)
        jend = lax.shift_right_logical(end + 15, 4)

        is_last = w == NWIN - 1

        def win_body(j, _, base=base, end=end, is_last=is_last, buf=buf):
            lane_pos = lanes + j * 16
            m = (lane_pos >= base) & (lane_pos < end)
            pk = plsc.load_gather(bpk_v, [lane_pos])
            lofs = jnp.bitwise_and(pk, WIN - 1)
            bp = lax.shift_right_logical(pk, LOG2W)
            if not is_last:
                c0 = plsc.load_gather(buf, [z16, lofs], mask=m)
                c1 = plsc.load_gather(buf, [o16, lofs], mask=m)
            else:
                m_in = m & (lofs < LAST_WIN)
                m_tl = m & (lofs >= LAST_WIN)
                tofs = lofs - LAST_WIN
                c0a = plsc.load_gather(buf, [z16, lofs], mask=m_in)
                c1a = plsc.load_gather(buf, [o16, lofs], mask=m_in)
                c0b = plsc.load_gather(tail_v, [z16, tofs], mask=m_tl)
                c1b = plsc.load_gather(tail_v, [o16, tofs], mask=m_tl)
                c0 = jnp.where(m_in, c0a, c0b)
                c1 = jnp.where(m_in, c1a, c1b)
            plsc.store_scatter(out_v, [z16, bp], c0, mask=m)
            plsc.store_scatter(out_v, [o16, bp], c1, mask=m)
            return ()

        lax.fori_loop(jstart, jend, win_body, ())
        if w + 2 < NWIN:
            cps.append(start_win(w + 2))

    pltpu.sync_copy(out_v, out_hbm.at[pl.ds(row0, E_PER_W)])


def kernel(inputs, v):
    idx = inputs.reshape(BATCH)
    wtail = v[TAIL:, :].T
    out_t = _gather(idx, v.T, wtail)
    return out_t.T
